# baseline (speedup 1.0000x reference)
import functools

import jax
import jax.numpy as jnp
from jax import lax
from jax.experimental import pallas as pl
from jax.experimental.pallas import tpu as pltpu
from jax.experimental.pallas import tpu_sc as plsc

B = 1024
NPL = 1024
L = 8
DEG = 16
EPL = NPL * DEG
NW = 32
BPW = B // NW
WBLK = NPL * BPW


def _sc_forward(x_flat, srcs, ws, bs):
    mesh = plsc.VectorSubcoreMesh(core_axis_name="c", subcore_axis_name="s")

    @functools.partial(
        pl.kernel,
        mesh=mesh,
        out_type=jax.ShapeDtypeStruct((NW * WBLK,), jnp.float32),
        scratch_types=[
            pltpu.VMEM((WBLK,), jnp.float32),
            pltpu.VMEM((WBLK,), jnp.float32),
            pltpu.VMEM((EPL,), jnp.int32),
            pltpu.VMEM((EPL,), jnp.float32),
            pltpu.VMEM((NPL + 16,), jnp.float32),
        ],
        compiler_params=pltpu.CompilerParams(needs_layout_passes=False),
    )
    def body(x_hbm, srcs_hbm, ws_hbm, bs_hbm, out_hbm, acts_a, acts_b, src_v,
             w_v, b_v):
        wid = lax.axis_index("s") * 2 + lax.axis_index("c")
        pltpu.sync_copy(x_hbm.at[pl.ds(wid * WBLK, WBLK)], acts_a)

        iota16 = lax.iota(jnp.int32, 16)
        row_off = iota16 * BPW

        bufs = [acts_a, acts_b]
        for l in range(L - 1):
            cur = bufs[l % 2]
            nxt = bufs[(l + 1) % 2]
            pltpu.sync_copy(srcs_hbm.at[pl.ds(l * EPL, EPL)], src_v)
            pltpu.sync_copy(ws_hbm.at[pl.ds(l * EPL, EPL)], w_v)
            pltpu.sync_copy(bs_hbm.at[pl.ds(l * NPL, NPL)],
                            b_v.at[pl.ds(0, NPL)])

            def group_body(g, _, cur=cur, nxt=nxt):
                n0 = g * 16
                ebase = n0 * DEG + iota16 * DEG
                bias16 = b_v[pl.ds(n0, 16)]
                addrs = []
                wks = []
                for k in range(DEG):
                    offs = ebase + k
                    sk = plsc.load_gather(src_v, [offs])
                    wk = plsc.load_gather(w_v, [offs])
                    addrs.append(sk * BPW)
                    wks.append(wk)
                out_idx0 = n0 * BPW + row_off

                def col_body(c, _):
                    acc = bias16
                    for k in range(DEG):
                        vals = plsc.load_gather(cur, [addrs[k] + c])
                        acc = acc + wks[k] * vals
                    acc = jnp.maximum(acc, 0.1 * acc)
                    plsc.store_scatter(nxt, [out_idx0 + c], acc)
                    return 0

                lax.fori_loop(0, BPW, col_body, 0)
                return 0

            lax.fori_loop(0, NPL // 16, group_body, 0)

        pltpu.sync_copy(bufs[(L - 1) % 2],
                        out_hbm.at[pl.ds(wid * WBLK, WBLK)])

    return body(x_flat, srcs, ws, bs)


def kernel(x, w, b, edge_src, edge_dst):
    del edge_dst
    srcs = (edge_src.reshape(L - 1, EPL) - (
        jnp.arange(L - 1, dtype=jnp.int32) * NPL)[:, None]).reshape(-1)
    bs = b[NPL:]
    x_flat = x.T.reshape(NPL, NW, BPW).transpose(1, 0, 2).reshape(-1)
    out_flat = _sc_forward(x_flat, srcs, w, bs)
    return out_flat.reshape(NW, NPL, BPW).transpose(1, 0, 2).reshape(NPL, B).T


# speedup vs baseline: 10.0295x; 10.0295x over previous
import functools

import jax
import jax.numpy as jnp
from jax import lax
from jax.experimental import pallas as pl
from jax.experimental.pallas import tpu as pltpu
from jax.experimental.pallas import tpu_sc as plsc

B = 1024
NPL = 1024
L = 8
DEG = 16
EPL = NPL * DEG
NW = 32
BPW = B // NW
WBLK = NPL * BPW


def _sc_forward(x_flat, srcs, ws, bs):
    mesh = plsc.VectorSubcoreMesh(core_axis_name="c", subcore_axis_name="s")

    @functools.partial(
        pl.kernel,
        mesh=mesh,
        out_type=jax.ShapeDtypeStruct((NW * WBLK,), jnp.float32),
        scratch_types=[
            pltpu.VMEM((WBLK,), jnp.float32),
            pltpu.VMEM((WBLK,), jnp.float32),
            pltpu.VMEM((EPL,), jnp.int32),
            pltpu.VMEM((EPL,), jnp.float32),
            pltpu.VMEM((NPL + 16,), jnp.float32),
        ],
        compiler_params=pltpu.CompilerParams(needs_layout_passes=False),
    )
    def body(x_hbm, srcs_hbm, ws_hbm, bs_hbm, out_hbm, acts_a, acts_b, src_v,
             w_v, b_v):
        wid = lax.axis_index("s") * 2 + lax.axis_index("c")
        pltpu.sync_copy(x_hbm.at[pl.ds(wid * WBLK, WBLK)], acts_a)

        bufs = [acts_a, acts_b]
        for l in range(L - 1):
            cur = bufs[l % 2]
            nxt = bufs[(l + 1) % 2]
            pltpu.sync_copy(srcs_hbm.at[pl.ds(l * EPL, EPL)], src_v)
            pltpu.sync_copy(ws_hbm.at[pl.ds(l * EPL, EPL)], w_v)
            pltpu.sync_copy(bs_hbm.at[pl.ds(l * NPL, NPL)],
                            b_v.at[pl.ds(0, NPL)])

            def node_body(n, cur=cur, nxt=nxt):
                e0 = n * DEG
                a16 = src_v[pl.ds(e0, DEG)]
                w16 = w_v[pl.ds(e0, DEG)]
                bn = b_v[pl.ds(n, 16)][0]
                parts0, parts1 = [], []
                for j in range(4):
                    p0 = p1 = None
                    for t in range(4):
                        k = j * 4 + t
                        r = a16[k]
                        wk = w16[k]
                        m0 = wk * cur[pl.ds(r, 16)]
                        m1 = wk * cur[pl.ds(r + 16, 16)]
                        p0 = m0 if p0 is None else p0 + m0
                        p1 = m1 if p1 is None else p1 + m1
                    parts0.append(p0)
                    parts1.append(p1)
                acc0 = (parts0[0] + parts0[1]) + (parts0[2] + parts0[3]) + bn
                acc1 = (parts1[0] + parts1[1]) + (parts1[2] + parts1[3]) + bn
                acc0 = jnp.maximum(acc0, 0.1 * acc0)
                acc1 = jnp.maximum(acc1, 0.1 * acc1)
                o0 = n * BPW
                nxt[pl.ds(o0, 16)] = acc0
                nxt[pl.ds(o0 + 16, 16)] = acc1

            plsc.parallel_loop(0, NPL, unroll=2)(node_body)

        pltpu.sync_copy(bufs[(L - 1) % 2],
                        out_hbm.at[pl.ds(wid * WBLK, WBLK)])

    return body(x_flat, srcs, ws, bs)


def kernel(x, w, b, edge_src, edge_dst):
    del edge_dst
    srcs = ((edge_src.reshape(L - 1, EPL) - (
        jnp.arange(L - 1, dtype=jnp.int32) * NPL)[:, None]) * BPW).reshape(-1)
    bs = b[NPL:]
    x_flat = x.T.reshape(NPL, NW, BPW).transpose(1, 0, 2).reshape(-1)
    out_flat = _sc_forward(x_flat, srcs, w, bs)
    return out_flat.reshape(NW, NPL, BPW).transpose(1, 0, 2).reshape(NPL, B).T
